# baseline (speedup 1.0000x reference)
import functools

import jax
import jax.numpy as jnp
from jax import lax
from jax.experimental import pallas as pl
from jax.experimental.pallas import tpu as pltpu
from jax.experimental.pallas import tpu_sc as plsc

PI = 3.141592653589793
GAMMA = 12.0
EPSILON = 2.0
EMBED_DIM = 64
D = 2 * EMBED_DIM
EMB_RANGE = (GAMMA + EPSILON) / EMBED_DIM
PHASE_SCALE = PI / EMB_RANGE

BATCH = 16384
HALF = BATCH // 2
NW = 32
CHUNK = 64
ROWS_PER_W = HALF // NW
CHUNKS_PER_W = ROWS_PER_W // CHUNK
LANES = 16


def _sc_gather_rotate(entity, cs_table, anc1_2d, anc2_2d, rel1_2d, rel2_2d):
    mesh = plsc.VectorSubcoreMesh(core_axis_name="c", subcore_axis_name="s")
    out_type = (
        jax.ShapeDtypeStruct((HALF, D), jnp.float32),
        jax.ShapeDtypeStruct((HALF, D), jnp.float32),
    )
    NB = 4

    @functools.partial(
        pl.kernel,
        out_type=out_type,
        mesh=mesh,
        scratch_types=[
            pltpu.VMEM((4, ROWS_PER_W), jnp.int32),
            pltpu.VMEM((NB, CHUNK, D), jnp.float32),
            pltpu.VMEM((NB, CHUNK, D), jnp.float32),
            pltpu.VMEM((NB, CHUNK, D), jnp.float32),
            pltpu.SemaphoreType.DMA((4,)),
            pltpu.SemaphoreType.DMA((NB,)),
            pltpu.SemaphoreType.DMA((NB,)),
            pltpu.SemaphoreType.DMA((NB,)),
        ],
    )
    def body(ent_hbm, cs_hbm, i1_hbm, i2_hbm, r1_hbm, r2_hbm,
             q1_out, q2_out, idx_v, ebuf, cbuf, qbuf,
             isem, esem, gsem, qsem):
        wid = lax.axis_index("s") * 2 + lax.axis_index("c")
        row0 = wid * ROWS_PER_W

        idx_copies = [
            pltpu.async_copy(h.at[pl.ds(row0, ROWS_PER_W)], idx_v.at[t],
                             isem.at[t])
            for t, h in enumerate((i1_hbm, i2_hbm, r1_hbm, r2_hbm))
        ]
        for h in idx_copies:
            h.wait()

        units = [(t, c) for t in range(2) for c in range(CHUNKS_PER_W)]
        outs = (q1_out, q2_out)
        n = len(units)

        def fire_pair(u):
            t, c = units[u]
            s = u % NB
            ge = pltpu.async_copy(ent_hbm.at[idx_v.at[t, pl.ds(c * CHUNK, CHUNK)]],
                                  ebuf.at[s], esem.at[s])
            gc = pltpu.async_copy(cs_hbm.at[idx_v.at[2 + t, pl.ds(c * CHUNK, CHUNK)]],
                                  cbuf.at[s], gsem.at[s])
            return ge, gc

        def rotate(s):
            def row(r, carry):
                for k in range(EMBED_DIM // LANES):
                    lo = k * LANES
                    hi = EMBED_DIM + k * LANES
                    are = ebuf[s, r, pl.ds(lo, LANES)]
                    aim = ebuf[s, r, pl.ds(hi, LANES)]
                    cr = cbuf[s, r, pl.ds(lo, LANES)]
                    sr = cbuf[s, r, pl.ds(hi, LANES)]
                    qbuf[s, r, pl.ds(lo, LANES)] = are * cr - aim * sr
                    qbuf[s, r, pl.ds(hi, LANES)] = are * sr + aim * cr
                return carry
            lax.fori_loop(0, CHUNK, row, 0)

        pairs = {u: fire_pair(u) for u in range(min(NB, n))}
        cps = {}
        for u in range(n):
            t, c = units[u]
            s = u % NB
            ge, gc = pairs[u]
            ge.wait()
            gc.wait()
            if u - NB >= 0:
                cps[u - NB].wait()
            rotate(s)
            cps[u] = pltpu.async_copy(
                qbuf.at[s],
                outs[t].at[pl.ds(wid * ROWS_PER_W + c * CHUNK, CHUNK)],
                qsem.at[s])
            if u + NB < n:
                pairs[u + NB] = fire_pair(u + NB)
        for u in range(max(0, n - NB), n):
            cps[u].wait()

    return body(entity, cs_table, anc1_2d, anc2_2d, rel1_2d, rel2_2d)


def _cs_table_body(rel_ref, cs_ref):
    ph = rel_ref[...] * PHASE_SCALE
    cs_ref[...] = jnp.concatenate([jnp.cos(ph), jnp.sin(ph)], axis=1)


def _tc_body(q1_ref, q2_ref, w1_ref, b1_ref, w2_ref, b2_ref, acc_ref,
             out_ref):
    del acc_ref
    w1 = w1_ref[...]
    w2 = w2_ref[...]
    b1 = b1_ref[...]
    b2 = b2_ref[...]

    def logits(q):
        h = jax.lax.dot_general(q, w1, (((1,), (1,)), ((), ())),
                                preferred_element_type=jnp.float32) + b1
        h = jnp.maximum(h, 0.0)
        return jax.lax.dot_general(h, w2, (((1,), (1,)), ((), ())),
                                   preferred_element_type=jnp.float32) + b2

    q1 = q1_ref[...]
    q2 = q2_ref[...]
    l1 = logits(q1)
    l2 = logits(q2)
    dl = l1 - l2
    e = jnp.exp(-jnp.abs(dl))
    w_hi = 1.0 / (1.0 + e)
    w_lo = 1.0 - w_hi
    first_hi = dl >= 0.0
    wq1 = jnp.where(first_hi, w_hi, w_lo)
    out_ref[...] = wq1 * q1 + (1.0 - wq1) * q2


def kernel(entity_embedding, relation_embedding, W1, b1, W2, b2,
           anc1_idx, anc2_idx, rel1_idx, rel2_idx):
    anc1_2d = anc1_idx.astype(jnp.int32)
    anc2_2d = anc2_idx.astype(jnp.int32)
    rel1_2d = rel1_idx.astype(jnp.int32)
    rel2_2d = rel2_idx.astype(jnp.int32)

    n_rel = relation_embedding.shape[0]
    cs_table = pl.pallas_call(
        _cs_table_body,
        out_shape=jax.ShapeDtypeStruct((n_rel, D), jnp.float32),
    )(relation_embedding)

    q1a, q2a = _sc_gather_rotate(entity_embedding, cs_table,
                                 anc1_2d[:HALF], anc2_2d[:HALF],
                                 rel1_2d[:HALF], rel2_2d[:HALF])
    q1b, q2b = _sc_gather_rotate(entity_embedding, cs_table,
                                 anc1_2d[HALF:], anc2_2d[HALF:],
                                 rel1_2d[HALF:], rel2_2d[HALF:])

    BS = 4096
    b1r = b1.reshape(1, D)
    b2r = b2.reshape(1, D)

    def tc_half(q1h, q2h, acc, half_idx):
        nblk = HALF // BS
        off = half_idx * nblk
        return pl.pallas_call(
            _tc_body,
            grid=(nblk,),
            in_specs=[
                pl.BlockSpec((BS, D), lambda i: (i, 0)),
                pl.BlockSpec((BS, D), lambda i: (i, 0)),
                pl.BlockSpec((D, D), lambda i: (0, 0)),
                pl.BlockSpec((1, D), lambda i: (0, 0)),
                pl.BlockSpec((D, D), lambda i: (0, 0)),
                pl.BlockSpec((1, D), lambda i: (0, 0)),
                pl.BlockSpec((8, D), lambda i: (0, 0)),
            ],
            out_specs=pl.BlockSpec((BS, D), lambda i: (i + off, 0)),
            out_shape=jax.ShapeDtypeStruct((BATCH, D), jnp.float32),
            input_output_aliases={6: 0},
        )(q1h, q2h, W1, b1r, W2, b2r, acc)

    acc = jnp.zeros((BATCH, D), jnp.float32)
    acc = tc_half(q1a, q2a, acc, 0)
    out = tc_half(q1b, q2b, acc, 1)
    return out


# speedup vs baseline: 1.0995x; 1.0995x over previous
import functools

import jax
import jax.numpy as jnp
from jax import lax
from jax.experimental import pallas as pl
from jax.experimental.pallas import tpu as pltpu
from jax.experimental.pallas import tpu_sc as plsc

PI = 3.141592653589793
GAMMA = 12.0
EPSILON = 2.0
EMBED_DIM = 64
D = 2 * EMBED_DIM
EMB_RANGE = (GAMMA + EPSILON) / EMBED_DIM
PHASE_SCALE = PI / EMB_RANGE

BATCH = 16384
NW = 32
CHUNK = 64
ROWS_PER_W = BATCH // NW
CHUNKS_PER_W = ROWS_PER_W // CHUNK
LANES = 16


def _sc_gather_rotate(entity, cs_table, anc1_2d, anc2_2d, rel1_2d, rel2_2d):
    mesh = plsc.VectorSubcoreMesh(core_axis_name="c", subcore_axis_name="s")
    out_type = (
        jax.ShapeDtypeStruct((BATCH, D), jnp.float32),
        jax.ShapeDtypeStruct((BATCH, D), jnp.float32),
    )
    NB = 5

    @functools.partial(
        pl.kernel,
        out_type=out_type,
        mesh=mesh,
        scratch_types=[
            pltpu.VMEM((4, ROWS_PER_W), jnp.int32),
            pltpu.VMEM((NB, CHUNK, D), jnp.float32),
            pltpu.VMEM((NB, CHUNK, D), jnp.float32),
            pltpu.VMEM((NB, CHUNK, D), jnp.float32),
            pltpu.SemaphoreType.DMA((4,)),
            pltpu.SemaphoreType.DMA((NB,)),
            pltpu.SemaphoreType.DMA((NB,)),
            pltpu.SemaphoreType.DMA((NB,)),
        ],
    )
    def body(ent_hbm, cs_hbm, i1_hbm, i2_hbm, r1_hbm, r2_hbm,
             q1_out, q2_out, idx_v, ebuf, cbuf, qbuf,
             isem, esem, gsem, qsem):
        wid = lax.axis_index("s") * 2 + lax.axis_index("c")
        row0 = wid * ROWS_PER_W

        idx_copies = [
            pltpu.async_copy(h.at[pl.ds(row0, ROWS_PER_W)], idx_v.at[t],
                             isem.at[t])
            for t, h in enumerate((i1_hbm, i2_hbm, r1_hbm, r2_hbm))
        ]
        for h in idx_copies:
            h.wait()

        units = [(t, c) for t in range(2) for c in range(CHUNKS_PER_W)]
        outs = (q1_out, q2_out)
        n = len(units)

        def fire_pair(u):
            t, c = units[u]
            s = u % NB
            ge = pltpu.async_copy(ent_hbm.at[idx_v.at[t, pl.ds(c * CHUNK, CHUNK)]],
                                  ebuf.at[s], esem.at[s])
            gc = pltpu.async_copy(cs_hbm.at[idx_v.at[2 + t, pl.ds(c * CHUNK, CHUNK)]],
                                  cbuf.at[s], gsem.at[s])
            return ge, gc

        def rotate(s):
            def row(r, carry):
                for k in range(EMBED_DIM // LANES):
                    lo = k * LANES
                    hi = EMBED_DIM + k * LANES
                    are = ebuf[s, r, pl.ds(lo, LANES)]
                    aim = ebuf[s, r, pl.ds(hi, LANES)]
                    cr = cbuf[s, r, pl.ds(lo, LANES)]
                    sr = cbuf[s, r, pl.ds(hi, LANES)]
                    qbuf[s, r, pl.ds(lo, LANES)] = are * cr - aim * sr
                    qbuf[s, r, pl.ds(hi, LANES)] = are * sr + aim * cr
                return carry
            lax.fori_loop(0, CHUNK, row, 0)

        pairs = {u: fire_pair(u) for u in range(min(NB, n))}
        cps = {}
        for u in range(n):
            t, c = units[u]
            s = u % NB
            ge, gc = pairs[u]
            ge.wait()
            gc.wait()
            if u - NB >= 0:
                cps[u - NB].wait()
            rotate(s)
            cps[u] = pltpu.async_copy(
                qbuf.at[s],
                outs[t].at[pl.ds(wid * ROWS_PER_W + c * CHUNK, CHUNK)],
                qsem.at[s])
            if u + NB < n:
                pairs[u + NB] = fire_pair(u + NB)
        for u in range(max(0, n - NB), n):
            cps[u].wait()

    return body(entity, cs_table, anc1_2d, anc2_2d, rel1_2d, rel2_2d)


def _cs_table_body(rel_ref, cs_ref):
    ph = rel_ref[...] * PHASE_SCALE
    cs_ref[...] = jnp.concatenate([jnp.cos(ph), jnp.sin(ph)], axis=1)


def _tc_body(q1_ref, q2_ref, w1_ref, b1_ref, w2_ref, b2_ref, out_ref):
    w1 = w1_ref[...]
    w2 = w2_ref[...]
    b1 = b1_ref[...]
    b2 = b2_ref[...]

    def logits(q):
        h = jax.lax.dot_general(q, w1, (((1,), (1,)), ((), ())),
                                preferred_element_type=jnp.float32) + b1
        h = jnp.maximum(h, 0.0)
        return jax.lax.dot_general(h, w2, (((1,), (1,)), ((), ())),
                                   preferred_element_type=jnp.float32) + b2

    q1 = q1_ref[...]
    q2 = q2_ref[...]
    l1 = logits(q1)
    l2 = logits(q2)
    dl = l1 - l2
    e = jnp.exp(-jnp.abs(dl))
    w_hi = 1.0 / (1.0 + e)
    w_lo = 1.0 - w_hi
    first_hi = dl >= 0.0
    wq1 = jnp.where(first_hi, w_hi, w_lo)
    out_ref[...] = wq1 * q1 + (1.0 - wq1) * q2


def kernel(entity_embedding, relation_embedding, W1, b1, W2, b2,
           anc1_idx, anc2_idx, rel1_idx, rel2_idx):
    anc1_2d = anc1_idx.astype(jnp.int32)
    anc2_2d = anc2_idx.astype(jnp.int32)
    rel1_2d = rel1_idx.astype(jnp.int32)
    rel2_2d = rel2_idx.astype(jnp.int32)

    n_rel = relation_embedding.shape[0]
    cs_table = pl.pallas_call(
        _cs_table_body,
        out_shape=jax.ShapeDtypeStruct((n_rel, D), jnp.float32),
    )(relation_embedding)

    q1, q2 = _sc_gather_rotate(entity_embedding, cs_table,
                               anc1_2d, anc2_2d, rel1_2d, rel2_2d)

    BS = 4096
    grid = (BATCH // BS,)
    out = pl.pallas_call(
        _tc_body,
        grid=grid,
        in_specs=[
            pl.BlockSpec((BS, D), lambda i: (i, 0)),
            pl.BlockSpec((BS, D), lambda i: (i, 0)),
            pl.BlockSpec((D, D), lambda i: (0, 0)),
            pl.BlockSpec((1, D), lambda i: (0, 0)),
            pl.BlockSpec((D, D), lambda i: (0, 0)),
            pl.BlockSpec((1, D), lambda i: (0, 0)),
        ],
        out_specs=pl.BlockSpec((BS, D), lambda i: (i, 0)),
        out_shape=jax.ShapeDtypeStruct((BATCH, D), jnp.float32),
    )(q1, q2, W1, b1.reshape(1, D), W2, b2.reshape(1, D))
    return out
